# Initial kernel scaffold; baseline (speedup 1.0000x reference)
import jax
import jax.numpy as jnp
from jax.experimental import pallas as pl


def kernel(x, w, state_space):
    raise NotImplementedError("write your pallas kernel here")


# speedup vs baseline: 47.2872x; 47.2872x over previous
import functools

import jax
import jax.numpy as jnp
from jax.experimental import pallas as pl

RADIUS = 4
TEMP = 2.0
NEG_FILL = jnp.finfo(jnp.float32).min


def _tile_kernel(x_ref, w_ref, ss_ref, out_ref, *, n_states):
    x = x_ref[...]
    w = w_ref[...]
    lo = ss_ref[0, 0]
    ls = ss_ref[0, 1] - ss_ref[0, 0]
    s = jnp.round((x - lo) / ls).astype(jnp.int32)
    gx = -(w * x)
    j = jax.lax.broadcasted_iota(jnp.int32, x.shape + (n_states,), 2)
    delta = j - s[:, :, None]
    adelta = jnp.abs(delta)
    mask = (adelta >= 1) & (adelta <= RADIUS)
    val = gx[:, :, None] * (delta.astype(jnp.float32) * (ls / TEMP))
    out_ref[...] = jnp.where(mask, val, NEG_FILL)


def kernel(x, w, state_space):
    B, D = x.shape
    NS = state_space.shape[0]
    DBLK = 512
    grid = (D // DBLK,)
    out = pl.pallas_call(
        functools.partial(_tile_kernel, n_states=NS),
        grid=grid,
        in_specs=[
            pl.BlockSpec((B, DBLK), lambda i: (0, i)),
            pl.BlockSpec((1, DBLK), lambda i: (0, i)),
            pl.BlockSpec((1, NS), lambda i: (0, 0)),
        ],
        out_specs=pl.BlockSpec((B, DBLK, NS), lambda i: (0, i, 0)),
        out_shape=jax.ShapeDtypeStruct((B, D, NS), jnp.float32),
    )(x, w.reshape(1, D), state_space.reshape(1, NS))
    return out.reshape(B, D * NS)


# speedup vs baseline: 90.8067x; 1.9203x over previous
import functools

import jax
import jax.numpy as jnp
from jax.experimental import pallas as pl

RADIUS = 4
TEMP = 2.0
NEG_FILL = jnp.finfo(jnp.float32).min


def _tile_kernel(x_ref, w_ref, ss_ref, out_ref, *, n_states):
    x = x_ref[...]
    w = w_ref[...]
    B, DBLK = x.shape
    lo = ss_ref[0, 0]
    ls = ss_ref[0, 1] - ss_ref[0, 0]
    s = jnp.round((x - lo) / ls).astype(jnp.int32)
    gx = -(w * x)
    j = jax.lax.broadcasted_iota(jnp.int32, (B, DBLK, n_states), 2)
    delta = j - s[:, :, None]
    adelta = jnp.abs(delta)
    mask = (adelta >= 1) & (adelta <= RADIUS)
    val = gx[:, :, None] * (delta.astype(jnp.float32) * (ls / TEMP))
    out_ref[...] = jnp.where(mask, val, NEG_FILL).reshape(B, DBLK * n_states)


def kernel(x, w, state_space):
    B, D = x.shape
    NS = state_space.shape[0]
    DBLK = 512
    grid = (D // DBLK,)
    out = pl.pallas_call(
        functools.partial(_tile_kernel, n_states=NS),
        grid=grid,
        in_specs=[
            pl.BlockSpec((B, DBLK), lambda i: (0, i)),
            pl.BlockSpec((1, DBLK), lambda i: (0, i)),
            pl.BlockSpec((1, NS), lambda i: (0, 0)),
        ],
        out_specs=pl.BlockSpec((B, DBLK * NS), lambda i: (0, i)),
        out_shape=jax.ShapeDtypeStruct((B, D * NS), jnp.float32),
    )(x, w.reshape(1, D), state_space.reshape(1, NS))
    return out


# speedup vs baseline: 319.6990x; 3.5207x over previous
import functools

import jax
import jax.numpy as jnp
from jax.experimental import pallas as pl

RADIUS = 4
TEMP = 2.0
NEG_FILL = jnp.finfo(jnp.float32).min


def _tile_kernel(x_ref, w_ref, ss_ref, sel_ref, out_ref, *, n_states):
    x = x_ref[...]
    w = w_ref[...]
    B, DBLK = x.shape
    LBLK = DBLK * n_states
    lo = ss_ref[0, 0]
    ls = ss_ref[0, 1] - ss_ref[0, 0]

    u = w * x
    u_hi = u.astype(jnp.bfloat16)
    u_lo = (u - u_hi.astype(jnp.float32)).astype(jnp.bfloat16)
    stack = jnp.concatenate(
        [x.astype(jnp.bfloat16), u_hi, u_lo], axis=0)
    rep = jnp.dot(stack, sel_ref[...],
                  preferred_element_type=jnp.float32)
    x_r = rep[:B]
    u_r = rep[B:2 * B] + rep[2 * B:]

    s_r = jnp.round((x_r - lo) / ls)
    jf = jax.lax.broadcasted_iota(jnp.int32, (1, LBLK), 1) % n_states
    delta = jf.astype(jnp.float32) - s_r
    adelta = jnp.abs(delta)
    mask = (adelta >= 1.0) & (adelta <= float(RADIUS))
    val = u_r * (delta * (-ls / TEMP))
    out_ref[...] = jnp.where(mask, val, NEG_FILL)


def kernel(x, w, state_space):
    B, D = x.shape
    NS = state_space.shape[0]
    DBLK = 128
    LBLK = DBLK * NS
    sel = (jnp.arange(LBLK, dtype=jnp.int32)[None, :] // NS
           == jnp.arange(DBLK, dtype=jnp.int32)[:, None]).astype(jnp.bfloat16)
    grid = (D // DBLK,)
    out = pl.pallas_call(
        functools.partial(_tile_kernel, n_states=NS),
        grid=grid,
        in_specs=[
            pl.BlockSpec((B, DBLK), lambda i: (0, i)),
            pl.BlockSpec((1, DBLK), lambda i: (0, i)),
            pl.BlockSpec((1, NS), lambda i: (0, 0)),
            pl.BlockSpec((DBLK, LBLK), lambda i: (0, 0)),
        ],
        out_specs=pl.BlockSpec((B, LBLK), lambda i: (0, i)),
        out_shape=jax.ShapeDtypeStruct((B, D * NS), jnp.float32),
    )(x, w.reshape(1, D), state_space.reshape(1, NS), sel)
    return out


# speedup vs baseline: 321.0693x; 1.0043x over previous
import functools

import jax
import jax.numpy as jnp
from jax.experimental import pallas as pl

RADIUS = 4
TEMP = 2.0
NEG_FILL = jnp.finfo(jnp.float32).min


def _tile_kernel(x_ref, w_ref, ss_ref, sel_ref, out_ref, *, n_states):
    x = x_ref[...]
    w = w_ref[...]
    B, DBLK = x.shape
    LBLK = DBLK * n_states
    lo = ss_ref[0, 0]
    ls = ss_ref[0, 1] - ss_ref[0, 0]

    s = jnp.round((x - lo) / ls)
    u = (w * x) * (-ls / TEMP)
    u_hi = u.astype(jnp.bfloat16)
    u_lo = (u - u_hi.astype(jnp.float32)).astype(jnp.bfloat16)
    stack = jnp.concatenate(
        [s.astype(jnp.bfloat16), u_hi, u_lo], axis=0)
    rep = jnp.dot(stack, sel_ref[...],
                  preferred_element_type=jnp.float32)
    s_r = rep[:B]
    u_r = rep[B:2 * B] + rep[2 * B:]

    jf = jax.lax.broadcasted_iota(jnp.int32, (1, LBLK), 1) % n_states
    delta = jf.astype(jnp.float32) - s_r
    adelta = jnp.abs(delta)
    mask = (adelta >= 1.0) & (adelta <= float(RADIUS))
    out_ref[...] = jnp.where(mask, u_r * delta, NEG_FILL)


def kernel(x, w, state_space):
    B, D = x.shape
    NS = state_space.shape[0]
    DBLK = 128
    LBLK = DBLK * NS
    sel = (jnp.arange(LBLK, dtype=jnp.int32)[None, :] // NS
           == jnp.arange(DBLK, dtype=jnp.int32)[:, None]).astype(jnp.bfloat16)
    grid = (D // DBLK,)
    out = pl.pallas_call(
        functools.partial(_tile_kernel, n_states=NS),
        grid=grid,
        in_specs=[
            pl.BlockSpec((B, DBLK), lambda i: (0, i)),
            pl.BlockSpec((1, DBLK), lambda i: (0, i)),
            pl.BlockSpec((1, NS), lambda i: (0, 0)),
            pl.BlockSpec((DBLK, LBLK), lambda i: (0, 0)),
        ],
        out_specs=pl.BlockSpec((B, LBLK), lambda i: (0, i)),
        out_shape=jax.ShapeDtypeStruct((B, D * NS), jnp.float32),
    )(x, w.reshape(1, D), state_space.reshape(1, NS), sel)
    return out


# speedup vs baseline: 376.8376x; 1.1737x over previous
import functools

import jax
import jax.numpy as jnp
from jax.experimental import pallas as pl

RADIUS = 4
TEMP = 2.0
NEG_FILL = jnp.finfo(jnp.float32).min


def _tile_kernel(x_ref, w_ref, ss_ref, sel_ref, out_ref, *, n_states):
    x = x_ref[...]
    w = w_ref[...]
    B, DBLK = x.shape
    LBLK = DBLK * n_states
    lo = ss_ref[0, 0]
    ls = ss_ref[0, 1] - ss_ref[0, 0]

    s = jnp.round((x - lo) / ls)
    u = (w * x) * (-ls / TEMP)
    u_hi = u.astype(jnp.bfloat16)
    u_lo = (u - u_hi.astype(jnp.float32)).astype(jnp.bfloat16)
    stack = jnp.concatenate(
        [s.astype(jnp.bfloat16), u_hi, u_lo], axis=0)
    rep = jnp.dot(stack, sel_ref[...],
                  preferred_element_type=jnp.float32)
    s_r = rep[:B]
    u_r = rep[B:2 * B] + rep[2 * B:]

    jf = jax.lax.broadcasted_iota(jnp.int32, (1, LBLK), 1) % n_states
    delta = jf.astype(jnp.float32) - s_r
    adelta = jnp.abs(delta)
    mask = (adelta >= 1.0) & (adelta <= float(RADIUS))
    out_ref[...] = jnp.where(mask, u_r * delta, NEG_FILL)


def kernel(x, w, state_space):
    B, D = x.shape
    NS = state_space.shape[0]
    DBLK = 256
    LBLK = DBLK * NS
    sel = (jnp.arange(LBLK, dtype=jnp.int32)[None, :] // NS
           == jnp.arange(DBLK, dtype=jnp.int32)[:, None]).astype(jnp.bfloat16)
    grid = (D // DBLK,)
    out = pl.pallas_call(
        functools.partial(_tile_kernel, n_states=NS),
        grid=grid,
        in_specs=[
            pl.BlockSpec((B, DBLK), lambda i: (0, i)),
            pl.BlockSpec((1, DBLK), lambda i: (0, i)),
            pl.BlockSpec((1, NS), lambda i: (0, 0)),
            pl.BlockSpec((DBLK, LBLK), lambda i: (0, 0)),
        ],
        out_specs=pl.BlockSpec((B, LBLK), lambda i: (0, i)),
        out_shape=jax.ShapeDtypeStruct((B, D * NS), jnp.float32),
    )(x, w.reshape(1, D), state_space.reshape(1, NS), sel)
    return out
